# Initial kernel scaffold; baseline (speedup 1.0000x reference)
import jax
import jax.numpy as jnp
from jax.experimental import pallas as pl


def kernel(x, edge_index, fn_gamma, fn_beta, Wp, bp, W1, b1, W2, b2, W3, b3, ng, nb, Wo, bo):
    raise NotImplementedError("write your pallas kernel here")


# speedup vs baseline: 2.4932x; 2.4932x over previous
import functools

import jax
import jax.numpy as jnp
from jax import lax
from jax.experimental import pallas as pl
from jax.experimental.pallas import tpu as pltpu
from jax.experimental.pallas import tpu_sc as plsc

_N = 10000
_E = 160000
_F = 256
_H = 256
_OUT = 128
_HH = 128
_BN = 2000
_GRID = _N // _BN
_NS = 16
_CH = 128
_EPC = ((_E // _NS + _CH - 1) // _CH) * _CH
_EPAD = _EPC * _NS
_NCH = _EPC // _CH
_ROWS_PT = 624
_ROWS_TAIL = _N - _ROWS_PT * (_NS - 1)
_ACC_ROWS = _N + 8
_EPS = 1e-5


def _mm(a, b):
    return jnp.dot(a, b, preferred_element_type=jnp.float32,
                   precision=jax.lax.Precision.HIGHEST)


def _sc_agg_body(x2, srcp, dstp, out, srcv, dstv, idxv, rows, acc, sem):
    c = lax.axis_index("c")
    s = lax.axis_index("s")
    pltpu.sync_copy(x2.at[pl.ds(c * _N + s * _ROWS_PT, _ROWS_PT)],
                    acc.at[pl.ds(s * _ROWS_PT, _ROWS_PT)])

    @pl.when(s == _NS - 1)
    def _():
        tail0 = _ROWS_PT * _NS
        pltpu.sync_copy(x2.at[pl.ds(c * _N + tail0, _N - tail0)],
                        acc.at[pl.ds(tail0, _N - tail0)])

    plsc.subcore_barrier()
    base0 = s * _EPC
    off = c * _N

    def chunk(e, carry):
        base = pl.multiple_of(base0 + e * _CH, _CH)
        pltpu.sync_copy(srcp.at[pl.ds(base, _CH)], srcv)
        pltpu.sync_copy(dstp.at[pl.ds(base, _CH)], dstv)
        for j in range(_CH // 16):
            idxv[pl.ds(j * 16, 16)] = srcv[pl.ds(j * 16, 16)] + off
        pltpu.async_copy(x2.at[idxv], rows, sem).wait()
        pltpu.sync_copy(rows, acc.at[dstv], add=True)
        return carry

    lax.fori_loop(0, _NCH, chunk, 0, unroll=False)
    plsc.subcore_barrier()
    pltpu.sync_copy(acc.at[pl.ds(s * _ROWS_PT, _ROWS_PT)],
                    out.at[pl.ds(c * _N + s * _ROWS_PT, _ROWS_PT)])

    @pl.when(s == _NS - 1)
    def _():
        tail0 = _ROWS_PT * _NS
        pltpu.sync_copy(acc.at[pl.ds(tail0, _N - tail0)],
                        out.at[pl.ds(c * _N + tail0, _N - tail0)])


@functools.cache
def _sc_agg_kernel():
    return pl.kernel(
        _sc_agg_body,
        out_type=jax.ShapeDtypeStruct((2 * _N, _HH), jnp.float32),
        mesh=plsc.VectorSubcoreMesh(core_axis_name="c", subcore_axis_name="s",
                                    num_cores=2, num_subcores=_NS),
        scratch_types=[
            pltpu.VMEM((_CH,), jnp.int32),
            pltpu.VMEM((_CH,), jnp.int32),
            pltpu.VMEM((_CH,), jnp.int32),
            pltpu.VMEM((_CH, _HH), jnp.float32),
            pltpu.VMEM_SHARED((_ACC_ROWS, _HH), jnp.float32),
            pltpu.SemaphoreType.DMA,
        ],
    )


def _sc_agg(x2, srcp, dstp):
    return _sc_agg_kernel()(x2, srcp, dstp)


def _stats_body(x_ref, st_ref):
    j = pl.program_id(0)
    xb = x_ref[...]
    s = jnp.concatenate([jnp.sum(xb, 0, keepdims=True),
                         jnp.sum(xb * xb, 0, keepdims=True)], axis=0)

    @pl.when(j == 0)
    def _():
        st_ref[...] = jnp.zeros_like(st_ref)

    st_ref[...] += s


_stats = pl.pallas_call(
    _stats_body,
    grid=(_GRID,),
    in_specs=[pl.BlockSpec((_BN, _F), lambda j: (j, 0))],
    out_specs=pl.BlockSpec((2, _F), lambda j: (0, 0)),
    out_shape=jax.ShapeDtypeStruct((2, _F), jnp.float32),
    compiler_params=pltpu.CompilerParams(
        dimension_semantics=("arbitrary",)),
    name="gcn_stats",
)


def _norm_affine(st_ref, g_ref, b_ref, n_rows):
    mu = st_ref[0:1, :] / n_rows
    var = st_ref[1:2, :] / n_rows - mu * mu
    sc = lax.rsqrt(var + _EPS) * g_ref[...]
    return mu, sc, b_ref[...]


def _proj_body(x_ref, st_ref, g_ref, b_ref, wp_ref, bp_ref, out_ref):
    mu, sc, beta = _norm_affine(st_ref, g_ref, b_ref, float(_N))
    xn = (x_ref[...] - mu) * sc + beta
    z = jnp.maximum(_mm(xn, wp_ref[...]) + bp_ref[...], 0.0)
    out_ref[0] = z[:, :_HH]
    out_ref[1] = z[:, _HH:]


_proj = pl.pallas_call(
    _proj_body,
    grid=(_GRID,),
    in_specs=[
        pl.BlockSpec((_BN, _F), lambda j: (j, 0)),
        pl.BlockSpec((2, _F), lambda j: (0, 0)),
        pl.BlockSpec((1, _F), lambda j: (0, 0)),
        pl.BlockSpec((1, _F), lambda j: (0, 0)),
        pl.BlockSpec((_F, _H), lambda j: (0, 0)),
        pl.BlockSpec((1, _H), lambda j: (0, 0)),
    ],
    out_specs=pl.BlockSpec((2, _BN, _HH), lambda j: (0, j, 0)),
    out_shape=jax.ShapeDtypeStruct((2, _N, _HH), jnp.float32),
    compiler_params=pltpu.CompilerParams(
        dimension_semantics=("arbitrary",)),
    name="gcn_proj",
)


def _mlp_body(a_ref, w1_ref, b1_ref, w2_ref, b2_ref, w3_ref, b3_ref,
              h3_ref, st_ref):
    j = pl.program_id(0)
    h = jnp.maximum(_mm(a_ref[0], w1_ref[:_HH, :])
                    + _mm(a_ref[1], w1_ref[_HH:, :]) + b1_ref[...], 0.0)
    h = jnp.maximum(_mm(h, w2_ref[...]) + b2_ref[...], 0.0)
    h = _mm(h, w3_ref[...]) + b3_ref[...]
    h3_ref[...] = h
    s = jnp.concatenate([jnp.sum(h, 0, keepdims=True),
                         jnp.sum(h * h, 0, keepdims=True)], axis=0)

    @pl.when(j == 0)
    def _():
        st_ref[...] = jnp.zeros_like(st_ref)

    st_ref[...] += s


_mlp = pl.pallas_call(
    _mlp_body,
    grid=(_GRID,),
    in_specs=[
        pl.BlockSpec((2, _BN, _HH), lambda j: (0, j, 0)),
        pl.BlockSpec((_H, _H), lambda j: (0, 0)),
        pl.BlockSpec((1, _H), lambda j: (0, 0)),
        pl.BlockSpec((_H, _H), lambda j: (0, 0)),
        pl.BlockSpec((1, _H), lambda j: (0, 0)),
        pl.BlockSpec((_H, _H), lambda j: (0, 0)),
        pl.BlockSpec((1, _H), lambda j: (0, 0)),
    ],
    out_specs=[
        pl.BlockSpec((_BN, _H), lambda j: (j, 0)),
        pl.BlockSpec((2, _H), lambda j: (0, 0)),
    ],
    out_shape=[
        jax.ShapeDtypeStruct((_N, _H), jnp.float32),
        jax.ShapeDtypeStruct((2, _H), jnp.float32),
    ],
    compiler_params=pltpu.CompilerParams(
        dimension_semantics=("arbitrary",)),
    name="gcn_mlp",
)


def _norm_body(h3_ref, st_ref, g_ref, b_ref, res_ref, out_ref, *, add_res):
    mu, sc, beta = _norm_affine(st_ref, g_ref, b_ref, float(_N))
    xh = jnp.maximum((h3_ref[...] - mu) * sc + beta, 0.0)
    x0 = xh[:, :_HH]
    x1 = xh[:, _HH:]
    if add_res:
        x0 = x0 + res_ref[0]
        x1 = x1 + res_ref[1]
    out_ref[0] = x0
    out_ref[1] = x1


def _make_norm(add_res):
    return pl.pallas_call(
        functools.partial(_norm_body, add_res=add_res),
        grid=(_GRID,),
        in_specs=[
            pl.BlockSpec((_BN, _H), lambda j: (j, 0)),
            pl.BlockSpec((2, _H), lambda j: (0, 0)),
            pl.BlockSpec((1, _H), lambda j: (0, 0)),
            pl.BlockSpec((1, _H), lambda j: (0, 0)),
            pl.BlockSpec((2, _BN, _HH), lambda j: (0, j, 0)),
        ],
        out_specs=pl.BlockSpec((2, _BN, _HH), lambda j: (0, j, 0)),
        out_shape=jax.ShapeDtypeStruct((2, _N, _HH), jnp.float32),
        compiler_params=pltpu.CompilerParams(
            dimension_semantics=("arbitrary",)),
        name="gcn_norm_res" if add_res else "gcn_norm",
    )


_norm_plain = _make_norm(False)
_norm_res = _make_norm(True)


def _final_body(x_ref, wo_ref, bo_ref, emb_ref, log_ref):
    j = pl.program_id(0)
    s = jnp.concatenate([jnp.sum(x_ref[0], 0, keepdims=True),
                         jnp.sum(x_ref[1], 0, keepdims=True)], axis=1)

    @pl.when(j == 0)
    def _():
        emb_ref[...] = jnp.zeros_like(emb_ref)

    emb_ref[...] += s

    @pl.when(j == _GRID - 1)
    def _():
        pooled = emb_ref[...] / float(_N)
        emb_ref[...] = pooled
        log_ref[...] = _mm(pooled, wo_ref[...]) + bo_ref[...]

    @pl.when(j < _GRID - 1)
    def _():
        log_ref[...] = jnp.zeros_like(log_ref)


_final = pl.pallas_call(
    _final_body,
    grid=(_GRID,),
    in_specs=[
        pl.BlockSpec((2, _BN, _HH), lambda j: (0, j, 0)),
        pl.BlockSpec((_H, _OUT), lambda j: (0, 0)),
        pl.BlockSpec((1, _OUT), lambda j: (0, 0)),
    ],
    out_specs=[
        pl.BlockSpec((1, _H), lambda j: (0, 0)),
        pl.BlockSpec((1, _OUT), lambda j: (0, 0)),
    ],
    out_shape=[
        jax.ShapeDtypeStruct((1, _H), jnp.float32),
        jax.ShapeDtypeStruct((1, _OUT), jnp.float32),
    ],
    compiler_params=pltpu.CompilerParams(
        dimension_semantics=("arbitrary",)),
    name="gcn_final",
)


def kernel(x, edge_index, fn_gamma, fn_beta, Wp, bp, W1, b1, W2, b2, W3, b3,
           ng, nb, Wo, bo):
    src = edge_index[0]
    dst = edge_index[1]
    pad = _EPAD - _E
    srcp = jnp.concatenate([src, jnp.zeros((pad,), jnp.int32)])
    dstp = jnp.concatenate([dst, jnp.full((pad,), _N, jnp.int32)])

    st = _stats(x)
    xs = _proj(x, st, fn_gamma.reshape(1, _F), fn_beta.reshape(1, _F),
               Wp, bp.reshape(1, _H))
    res = xs
    L = W1.shape[0]
    num_convs = 2
    for i in range(L):
        for k in range(num_convs):
            agg = _sc_agg(xs.reshape(2 * _N, _HH), srcp, dstp)
            h3, st = _mlp(agg.reshape(2, _N, _HH),
                          W1[i], b1[i].reshape(1, _H),
                          W2[i], b2[i].reshape(1, _H),
                          W3[i], b3[i].reshape(1, _H))
            if k == num_convs - 1:
                xs = _norm_res(h3, st, ng[i].reshape(1, _H),
                               nb[i].reshape(1, _H), res)
            else:
                xs = _norm_plain(h3, st, ng[i].reshape(1, _H),
                                 nb[i].reshape(1, _H), res)
        res = xs
    emb, logits = _final(xs, Wo, bo.reshape(1, _OUT))
    return emb.reshape(_H), logits.reshape(_OUT)
